# baseline (speedup 1.0000x reference)
import functools

import jax
import jax.numpy as jnp
from jax import lax
from jax.experimental import pallas as pl
from jax.experimental.pallas import tpu as pltpu
from jax.experimental.pallas import tpu_sc as plsc

_K = 5
_R = 1024
_INV_STEP = 1.0 / (_R - 1)

_NW = 32
_CPW = _R // _NW
_L = 16
_NVPW = _CPW // _L

_NB_TC = _R // 128
_PAD = 1024
_SC_TILES = 3

_NET = (
    (0, 1, False), (2, 3, False), (4, 5, False), (6, 7, False),
    (0, 2, False), (1, 3, False), (4, 6, False), (5, 7, False),
    (1, 2, False), (5, 6, False),
    (0, 4, False), (1, 5, False), (2, 6, True), (3, 7, True),
    (2, 4, False), (3, 5, True),
    (1, 2, False), (3, 4, False),
)


@functools.lru_cache(maxsize=None)
def _sc_call(n):
    mesh = plsc.VectorSubcoreMesh(core_axis_name="c", subcore_axis_name="s")
    n_sc = _SC_TILES * _PAD

    @functools.partial(
        pl.kernel,
        mesh=mesh,
        out_type=jax.ShapeDtypeStruct((_K, _R), jnp.float32),
        scratch_types=[
            pltpu.VMEM((n,), jnp.float32),
            pltpu.VMEM((n,), jnp.float32),
            pltpu.VMEM((_K, _CPW), jnp.float32),
        ],
    )
    def body(birth_hbm, death_hbm, out_hbm, b_v, d_v, o_v):
        wid = lax.axis_index("s") * 2 + lax.axis_index("c")
        pltpu.sync_copy(birth_hbm, b_v)
        pltpu.sync_copy(death_hbm, d_v)

        def red(i, carry):
            mn, mx = carry
            return (jnp.minimum(mn, b_v[pl.ds(i * _L, _L)]),
                    jnp.maximum(mx, d_v[pl.ds(i * _L, _L)]))

        mn0 = jnp.full((_L,), jnp.inf, jnp.float32)
        mx0 = jnp.full((_L,), -jnp.inf, jnp.float32)
        mn, mx = lax.fori_loop(0, n // _L, red, (mn0, mx0))
        minb = mn[0]
        maxd = mx[0]
        for i in range(1, _L):
            minb = jnp.minimum(minb, mn[i])
            maxd = jnp.maximum(maxd, mx[i])
        step = (maxd - minb) * jnp.float32(_INV_STEP)

        lane = lax.iota(jnp.int32, _L).astype(jnp.float32)
        base = (wid * _CPW).astype(jnp.float32)
        ts = tuple(minb + (base + jnp.float32(h * _L) + lane) * step
                   for h in range(_NVPW))

        init = (jnp.zeros((_L,), jnp.float32),) * (_K * _NVPW)

        def chunk_body(c, m):
            bv = b_v[pl.ds(c * _L, _L)]
            dv = d_v[pl.ds(c * _L, _L)]
            m = list(m)
            for half in range(2):
                bs = [bv[half * 8 + j] for j in range(8)]
                ds_ = [dv[half * 8 + j] for j in range(8)]
                for h in range(_NVPW):
                    t = ts[h]
                    vs = [jnp.minimum(t - bs[j], ds_[j] - t) for j in range(8)]
                    for a, b, maxonly in _NET:
                        hi = jnp.maximum(vs[a], vs[b])
                        if not maxonly:
                            vs[b] = jnp.minimum(vs[a], vs[b])
                        vs[a] = hi
                    for j in range(_K):
                        v = vs[j]
                        for q in range(j, _K):
                            mi = m[q * _NVPW + h]
                            m[q * _NVPW + h] = jnp.maximum(mi, v)
                            v = jnp.minimum(mi, v)
            return tuple(m)

        per_sub = n // 8 // _L
        sc_sub = n_sc // 8 // _L
        m = init
        for s in range(8):
            m = lax.fori_loop(s * per_sub, s * per_sub + sc_sub,
                              chunk_body, m)

        for i in range(_K):
            for h in range(_NVPW):
                o_v[i, pl.ds(h * _L, _L)] = m[i * _NVPW + h]
        for i in range(_K):
            pltpu.sync_copy(o_v.at[i],
                            out_hbm.at[i, pl.ds(wid * _CPW, _CPW)])

    return body


def _tc_topk_body(b_ref, d_ref, o_ref):
    minb = jnp.min(b_ref[...])
    maxd = jnp.max(d_ref[...])
    step = (maxd - minb) * jnp.float32(_INV_STEP)
    lanef = lax.broadcasted_iota(jnp.int32, (8, 128), 1).astype(jnp.float32)
    ts = [minb + (jnp.float32(blk * 128) + lanef) * step
          for blk in range(_NB_TC)]

    ntile = b_ref.shape[1] // 128

    def make_tile_body(blks):
        def tile_body(g, m):
            off = pl.multiple_of(g * 128, 128)
            bt = b_ref[:, pl.ds(off, 128)]
            dt = d_ref[:, pl.ds(off, 128)]
            m = list(m)
            for ub in range(16):
                b8s = [lax.slice(bt, (0, ub * 8 + j), (8, ub * 8 + j + 1))
                       for j in range(8)]
                d8s = [lax.slice(dt, (0, ub * 8 + j), (8, ub * 8 + j + 1))
                       for j in range(8)]
                for bi, blk in enumerate(blks):
                    t = ts[blk]
                    vs = [jnp.minimum(t - b8s[j], d8s[j] - t)
                          for j in range(8)]
                    for a, b, maxonly in _NET:
                        hi = jnp.maximum(vs[a], vs[b])
                        if not maxonly:
                            vs[b] = jnp.minimum(vs[a], vs[b])
                        vs[a] = hi
                    for j in range(_K):
                        v = vs[j]
                        for q in range(j, _K):
                            mi = m[bi * _K + q]
                            m[bi * _K + q] = jnp.maximum(mi, v)
                            v = jnp.minimum(mi, v)
            return tuple(m)

        return tile_body

    t0 = _SC_TILES * _PAD // 1024
    init4 = (jnp.zeros((8, 128), jnp.float32),) * (4 * _K)
    m_lo = lax.fori_loop(t0, ntile, make_tile_body((0, 1, 2, 3)), init4)
    m_hi = lax.fori_loop(t0, ntile, make_tile_body((4, 5, 6, 7)), init4)
    m = list(m_lo) + list(m_hi)

    for blk in range(_NB_TC):
        fin = [jnp.zeros((1, 128), jnp.float32) for _ in range(_K)]
        for s in range(8):
            for i in range(_K):
                v = lax.slice(m[blk * _K + i], (s, 0), (s + 1, 128))
                for q in range(i, _K):
                    fq = fin[q]
                    fin[q] = jnp.maximum(fq, v)
                    v = jnp.minimum(fq, v)
        for i in range(_K):
            o_ref[pl.ds(i, 1), pl.ds(blk * 128, 128)] = fin[i]


def _tc_merge_body(a_ref, b_ref, o_ref):
    fin = [a_ref[pl.ds(i, 1), :] for i in range(_K)]
    for i in range(_K):
        v = b_ref[pl.ds(i, 1), :]
        for q in range(i, _K):
            fq = fin[q]
            fin[q] = jnp.maximum(fq, v)
            v = jnp.minimum(fq, v)
    for i in range(_K):
        o_ref[pl.ds(i, 1), :] = fin[i]


def _tc_call(n):
    topk = pl.pallas_call(
        _tc_topk_body,
        out_shape=jax.ShapeDtypeStruct((_K, _R), jnp.float32),
    )
    merge = pl.pallas_call(
        _tc_merge_body,
        out_shape=jax.ShapeDtypeStruct((_K, _R), jnp.float32),
    )

    def run(birth, death):
        bt = birth.reshape(8, -1)
        dt = death.reshape(8, -1)
        return topk(bt, dt), merge

    return run


def kernel(pairs):
    n = ((pairs.shape[0] + _PAD - 1) // _PAD) * _PAD
    npad = n - pairs.shape[0]
    birth = jnp.pad(pairs[:, 0], (0, npad), constant_values=jnp.inf)
    death = jnp.pad(pairs[:, 1], (0, npad), constant_values=-jnp.inf)
    out_sc = _sc_call(n)(birth, death)
    out_tc, merge = _tc_call(n)(birth, death)
    return merge(out_tc, out_sc)


# speedup vs baseline: 1.0248x; 1.0248x over previous
import functools

import jax
import jax.numpy as jnp
from jax import lax
from jax.experimental import pallas as pl
from jax.experimental.pallas import tpu as pltpu
from jax.experimental.pallas import tpu_sc as plsc

_K = 5
_R = 1024
_INV_STEP = 1.0 / (_R - 1)

_NW = 32
_CPW = _R // _NW
_L = 16
_NVPW = _CPW // _L

_NB_TC = _R // 128
_PAD = 1024
_SC_TILES = 3

_NET = (
    (0, 1, False), (2, 3, False), (4, 5, False), (6, 7, False),
    (0, 2, False), (1, 3, False), (4, 6, False), (5, 7, False),
    (1, 2, False), (5, 6, False),
    (0, 4, False), (1, 5, False), (2, 6, True), (3, 7, True),
    (2, 4, False), (3, 5, True),
    (1, 2, False), (3, 4, False),
)


@functools.lru_cache(maxsize=None)
def _sc_call(n):
    mesh = plsc.VectorSubcoreMesh(core_axis_name="c", subcore_axis_name="s")
    n_sc = _SC_TILES * _PAD

    @functools.partial(
        pl.kernel,
        mesh=mesh,
        out_type=jax.ShapeDtypeStruct((_K, _R), jnp.float32),
        scratch_types=[
            pltpu.VMEM((n,), jnp.float32),
            pltpu.VMEM((n,), jnp.float32),
            pltpu.VMEM((_K, _CPW), jnp.float32),
        ],
    )
    def body(birth_hbm, death_hbm, out_hbm, b_v, d_v, o_v):
        wid = lax.axis_index("s") * 2 + lax.axis_index("c")
        pltpu.sync_copy(birth_hbm, b_v)
        pltpu.sync_copy(death_hbm, d_v)

        def red(i, carry):
            mn, mx = carry
            return (jnp.minimum(mn, b_v[pl.ds(i * _L, _L)]),
                    jnp.maximum(mx, d_v[pl.ds(i * _L, _L)]))

        mn0 = jnp.full((_L,), jnp.inf, jnp.float32)
        mx0 = jnp.full((_L,), -jnp.inf, jnp.float32)
        mn, mx = lax.fori_loop(0, n // _L, red, (mn0, mx0))
        minb = mn[0]
        maxd = mx[0]
        for i in range(1, _L):
            minb = jnp.minimum(minb, mn[i])
            maxd = jnp.maximum(maxd, mx[i])
        step = (maxd - minb) * jnp.float32(_INV_STEP)

        lane = lax.iota(jnp.int32, _L).astype(jnp.float32)
        base = (wid * _CPW).astype(jnp.float32)
        ts = tuple(minb + (base + jnp.float32(h * _L) + lane) * step
                   for h in range(_NVPW))

        init = (jnp.zeros((_L,), jnp.float32),) * (_K * _NVPW)

        def chunk_body(c, m):
            bv = b_v[pl.ds(c * _L, _L)]
            dv = d_v[pl.ds(c * _L, _L)]
            m = list(m)
            for half in range(2):
                bs = [bv[half * 8 + j] for j in range(8)]
                ds_ = [dv[half * 8 + j] for j in range(8)]
                for h in range(_NVPW):
                    t = ts[h]
                    vs = [jnp.minimum(t - bs[j], ds_[j] - t) for j in range(8)]
                    for a, b, maxonly in _NET:
                        hi = jnp.maximum(vs[a], vs[b])
                        if not maxonly:
                            vs[b] = jnp.minimum(vs[a], vs[b])
                        vs[a] = hi
                    for j in range(_K):
                        v = vs[j]
                        for q in range(j, _K):
                            mi = m[q * _NVPW + h]
                            m[q * _NVPW + h] = jnp.maximum(mi, v)
                            v = jnp.minimum(mi, v)
            return tuple(m)

        per_sub = n // 8 // _L
        sc_sub = n_sc // 8 // _L
        m = init
        for s in range(8):
            m = lax.fori_loop(s * per_sub, s * per_sub + sc_sub,
                              chunk_body, m)

        for i in range(_K):
            for h in range(_NVPW):
                o_v[i, pl.ds(h * _L, _L)] = m[i * _NVPW + h]
        for i in range(_K):
            pltpu.sync_copy(o_v.at[i],
                            out_hbm.at[i, pl.ds(wid * _CPW, _CPW)])

    return body


def _tc_topk_body(b_ref, d_ref, o_ref):
    minb = jnp.min(b_ref[...])
    maxd = jnp.max(d_ref[...])
    step = (maxd - minb) * jnp.float32(_INV_STEP)
    lanef = lax.broadcasted_iota(jnp.int32, (8, 128), 1).astype(jnp.float32)
    ts = [minb + (jnp.float32(blk * 128) + lanef) * step
          for blk in range(_NB_TC)]

    ntile = b_ref.shape[1] // 128

    def make_tile_body(blks):
        def tile_body(g, m):
            off = pl.multiple_of(g * 128, 128)
            bt = b_ref[:, pl.ds(off, 128)]
            dt = d_ref[:, pl.ds(off, 128)]
            m = list(m)
            for ub in range(16):
                b8s = [lax.slice(bt, (0, ub * 8 + j), (8, ub * 8 + j + 1))
                       for j in range(8)]
                d8s = [lax.slice(dt, (0, ub * 8 + j), (8, ub * 8 + j + 1))
                       for j in range(8)]
                for bi, blk in enumerate(blks):
                    t = ts[blk]
                    vs = [jnp.minimum(t - b8s[j], d8s[j] - t)
                          for j in range(8)]
                    for a, b, maxonly in _NET:
                        hi = jnp.maximum(vs[a], vs[b])
                        if not maxonly:
                            vs[b] = jnp.minimum(vs[a], vs[b])
                        vs[a] = hi
                    for j in range(_K):
                        v = vs[j]
                        for q in range(j, _K):
                            mi = m[bi * _K + q]
                            m[bi * _K + q] = jnp.maximum(mi, v)
                            v = jnp.minimum(mi, v)
            return tuple(m)

        return tile_body

    t0 = _SC_TILES * _PAD // 1024
    init = (jnp.zeros((8, 128), jnp.float32),) * (_NB_TC * _K)
    m = lax.fori_loop(t0, ntile, make_tile_body(tuple(range(_NB_TC))), init)
    m = list(m)

    for blk in range(_NB_TC):
        fin = [jnp.zeros((1, 128), jnp.float32) for _ in range(_K)]
        for s in range(8):
            for i in range(_K):
                v = lax.slice(m[blk * _K + i], (s, 0), (s + 1, 128))
                for q in range(i, _K):
                    fq = fin[q]
                    fin[q] = jnp.maximum(fq, v)
                    v = jnp.minimum(fq, v)
        for i in range(_K):
            o_ref[pl.ds(i, 1), pl.ds(blk * 128, 128)] = fin[i]


def _tc_merge_body(a_ref, b_ref, o_ref):
    fin = [a_ref[pl.ds(i, 1), :] for i in range(_K)]
    for i in range(_K):
        v = b_ref[pl.ds(i, 1), :]
        for q in range(i, _K):
            fq = fin[q]
            fin[q] = jnp.maximum(fq, v)
            v = jnp.minimum(fq, v)
    for i in range(_K):
        o_ref[pl.ds(i, 1), :] = fin[i]


def _tc_call(n):
    topk = pl.pallas_call(
        _tc_topk_body,
        out_shape=jax.ShapeDtypeStruct((_K, _R), jnp.float32),
    )
    merge = pl.pallas_call(
        _tc_merge_body,
        out_shape=jax.ShapeDtypeStruct((_K, _R), jnp.float32),
    )

    def run(birth, death):
        bt = birth.reshape(8, -1)
        dt = death.reshape(8, -1)
        return topk(bt, dt), merge

    return run


def kernel(pairs):
    n = ((pairs.shape[0] + _PAD - 1) // _PAD) * _PAD
    npad = n - pairs.shape[0]
    birth = jnp.pad(pairs[:, 0], (0, npad), constant_values=jnp.inf)
    death = jnp.pad(pairs[:, 1], (0, npad), constant_values=-jnp.inf)
    out_sc = _sc_call(n)(birth, death)
    out_tc, merge = _tc_call(n)(birth, death)
    return merge(out_tc, out_sc)
